# Initial kernel scaffold; baseline (speedup 1.0000x reference)
import jax
import jax.numpy as jnp
from jax.experimental import pallas as pl


def kernel(sequence, ngrams, word_table, ngram_table, Wn0, bn0, Wl0, bl0, Wg0, bg0, Wn1, bn1, Wl1, bl1, Wg1, bg1, Wo, bo):
    raise NotImplementedError("write your pallas kernel here")


# speedup vs baseline: 3.6263x; 3.6263x over previous
import functools

import jax
import jax.numpy as jnp
from jax import lax
from jax.experimental import pallas as pl
from jax.experimental.pallas import tpu as pltpu
from jax.experimental.pallas import tpu_sc as plsc

B = 4096
S = 200
D = 64
SIZE = 2 * D
CLASSES = 10
NC = 2
NS = 16
NW = NC * NS
BPW = B // NW
WIN = 40
NWIN = S // WIN
LANES = 16
NCH = D // LANES

_mesh = plsc.VectorSubcoreMesh(core_axis_name="c", subcore_axis_name="s")


@functools.partial(
    pl.kernel,
    out_type=[
        jax.ShapeDtypeStruct((B, D), jnp.float32),
        jax.ShapeDtypeStruct((B, D), jnp.float32),
    ],
    mesh=_mesh,
    compiler_params=pltpu.CompilerParams(use_tc_tiling_on_sc=False),
    scratch_types=[
        pltpu.VMEM((BPW, NWIN, WIN), jnp.int32),
        pltpu.VMEM((S, D), jnp.float32),
        pltpu.VMEM((S, D), jnp.float32),
        pltpu.VMEM((BPW, D), jnp.float32),
        pltpu.SemaphoreType.DMA,
        pltpu.SemaphoreType.DMA,
    ],
)
def _pool(seq_hbm, ngr_hbm, wt_hbm, nt_hbm, outw_hbm, outn_hbm,
          idx_v, rows_a, rows_b, out_v, sem_a, sem_b):
    wid = lax.axis_index("s") * NC + lax.axis_index("c")
    base = wid * BPW

    def fire(tab_hbm, elem, buf, sem):
        for j in range(NWIN):
            pltpu.async_copy(
                tab_hbm.at[idx_v.at[elem, j]],
                buf.at[pl.ds(j * WIN, WIN)],
                sem,
            )

    def drain(tab_hbm, buf, sem):
        pltpu.make_async_copy(tab_hbm.at[pl.ds(0, S)], buf, sem).wait()

    def accumulate(buf, elem):
        zero = jnp.zeros((LANES,), jnp.float32)

        def chunk(k, accs):
            r0 = k * 8
            vals = [[buf[r0 + r, pl.ds(c * LANES, LANES)] for c in range(NCH)]
                    for r in range(8)]
            nxt = []
            for c in range(NCH):
                s = ((vals[0][c] + vals[1][c]) + (vals[2][c] + vals[3][c])) \
                    + ((vals[4][c] + vals[5][c]) + (vals[6][c] + vals[7][c]))
                nxt.append(accs[c] + s)
            return tuple(nxt)

        accs = lax.fori_loop(0, S // 8, chunk, (zero,) * NCH)
        for c in range(NCH):
            out_v[elem, pl.ds(c * LANES, LANES)] = accs[c] * jnp.float32(1.0 / S)

    for tok_hbm, tab_hbm, out_hbm in ((seq_hbm, wt_hbm, outw_hbm),
                                      (ngr_hbm, nt_hbm, outn_hbm)):
        pltpu.sync_copy(tok_hbm.at[pl.ds(base, BPW)], idx_v)
        fire(tab_hbm, 0, rows_a, sem_a)
        fire(tab_hbm, 1, rows_b, sem_b)

        def elem_pair(i, carry):
            e = 2 * i
            drain(tab_hbm, rows_a, sem_a)
            accumulate(rows_a, e)

            @pl.when(e + 2 < BPW)
            def _():
                fire(tab_hbm, e + 2, rows_a, sem_a)

            drain(tab_hbm, rows_b, sem_b)
            accumulate(rows_b, e + 1)

            @pl.when(e + 3 < BPW)
            def _():
                fire(tab_hbm, e + 3, rows_b, sem_b)

            return carry

        lax.fori_loop(0, BPW // 2, elem_pair, 0)
        pltpu.sync_copy(out_v, out_hbm.at[pl.ds(base, BPW)])


_DN = (((1,), (1,)), ((), ()))


def _mm(x, w_ref):
    return lax.dot_general(x, w_ref[...], _DN,
                           precision=lax.Precision.HIGHEST,
                           preferred_element_type=jnp.float32)


def _mlp_body(xw_ref, xn_ref,
              wn0, bn0, wl0, bl0, wg0, bg0,
              wn1, bn1, wl1, bl1, wg1, bg1,
              wo, bo, out_ref):
    x = jnp.concatenate([xw_ref[...], xn_ref[...]], axis=1)
    for wn, bn, wl, bl, wg, bg in ((wn0, bn0, wl0, bl0, wg0, bg0),
                                   (wn1, bn1, wl1, bl1, wg1, bg1)):
        gate = jax.nn.sigmoid(_mm(x, wg) + bg[...])
        nonlinear = jax.nn.relu(_mm(x, wn) + bn[...])
        linear = _mm(x, wl) + bl[...]
        x = gate * nonlinear + (1.0 - gate) * linear
    out_ref[...] = _mm(x, wo) + bo[...]


_mlp = pl.pallas_call(
    _mlp_body,
    out_shape=jax.ShapeDtypeStruct((B, CLASSES), jnp.float32),
)


def kernel(sequence, ngrams, word_table, ngram_table,
           Wn0, bn0, Wl0, bl0, Wg0, bg0,
           Wn1, bn1, Wl1, bl1, Wg1, bg1,
           Wo, bo):
    seq = sequence.astype(jnp.int32).reshape(B, NWIN, WIN)
    ngr = ngrams.astype(jnp.int32).reshape(B, NWIN, WIN)
    embw, embn = _pool(seq, ngr, word_table, ngram_table)
    return _mlp(embw, embn,
                Wn0, bn0.reshape(1, SIZE), Wl0, bl0.reshape(1, SIZE),
                Wg0, bg0.reshape(1, SIZE),
                Wn1, bn1.reshape(1, SIZE), Wl1, bl1.reshape(1, SIZE),
                Wg1, bg1.reshape(1, SIZE),
                Wo, bo.reshape(1, CLASSES))


# speedup vs baseline: 4.0239x; 1.1096x over previous
import functools

import jax
import jax.numpy as jnp
from jax import lax
from jax.experimental import pallas as pl
from jax.experimental.pallas import tpu as pltpu
from jax.experimental.pallas import tpu_sc as plsc

B = 4096
S = 200
D = 64
SIZE = 2 * D
CLASSES = 10
NC = 2
NS = 16
NW = NC * NS
BPW = B // NW
WIN = 40
NWIN = S // WIN
LANES = 16
NCH = D // LANES

_mesh = plsc.VectorSubcoreMesh(core_axis_name="c", subcore_axis_name="s")


@functools.partial(
    pl.kernel,
    out_type=jax.ShapeDtypeStruct((B, D), jnp.float32),
    mesh=_mesh,
    compiler_params=pltpu.CompilerParams(use_tc_tiling_on_sc=False),
    scratch_types=[
        pltpu.VMEM((BPW, NWIN, WIN), jnp.int32),
        pltpu.VMEM((S, D), jnp.float32),
        pltpu.VMEM((S, D), jnp.float32),
        pltpu.VMEM((BPW, D), jnp.float32),
        pltpu.SemaphoreType.DMA,
        pltpu.SemaphoreType.DMA,
    ],
)
def _pool(tok_hbm, tab_hbm, out_hbm, idx_v, rows_a, rows_b, out_v,
          sem_a, sem_b):
    wid = lax.axis_index("s") * NC + lax.axis_index("c")
    base = wid * BPW

    def fire(elem, buf, sem):
        for j in range(NWIN):
            pltpu.async_copy(
                tab_hbm.at[idx_v.at[elem, j]],
                buf.at[pl.ds(j * WIN, WIN)],
                sem,
            )

    def drain(buf, sem):
        pltpu.make_async_copy(tab_hbm.at[pl.ds(0, S)], buf, sem).wait()

    def accumulate(buf, elem):
        zero = jnp.zeros((LANES,), jnp.float32)

        def chunk(k, accs):
            r0 = k * 8
            vals = [[buf[r0 + r, pl.ds(c * LANES, LANES)] for c in range(NCH)]
                    for r in range(8)]
            nxt = []
            for c in range(NCH):
                s = ((vals[0][c] + vals[1][c]) + (vals[2][c] + vals[3][c])) \
                    + ((vals[4][c] + vals[5][c]) + (vals[6][c] + vals[7][c]))
                nxt.append(accs[c] + s)
            return tuple(nxt)

        accs = lax.fori_loop(0, S // 8, chunk, (zero,) * NCH)
        for c in range(NCH):
            out_v[elem, pl.ds(c * LANES, LANES)] = accs[c] * jnp.float32(1.0 / S)

    pltpu.sync_copy(tok_hbm.at[pl.ds(base, BPW)], idx_v)
    fire(0, rows_a, sem_a)
    fire(1, rows_b, sem_b)

    def elem_pair(i, carry):
        e = 2 * i
        drain(rows_a, sem_a)
        accumulate(rows_a, e)

        @pl.when(e + 2 < BPW)
        def _():
            fire(e + 2, rows_a, sem_a)

        drain(rows_b, sem_b)
        accumulate(rows_b, e + 1)

        @pl.when(e + 3 < BPW)
        def _():
            fire(e + 3, rows_b, sem_b)

        return carry

    lax.fori_loop(0, BPW // 2, elem_pair, 0)
    pltpu.sync_copy(out_v, out_hbm.at[pl.ds(base, BPW)])


_DN = (((1,), (1,)), ((), ()))


def _mm(x, w_ref):
    return lax.dot_general(x, w_ref[...], _DN,
                           precision=lax.Precision.HIGHEST,
                           preferred_element_type=jnp.float32)


def _mlp_body(xw_ref, xn_ref,
              wn0, bn0, wl0, bl0, wg0, bg0,
              wn1, bn1, wl1, bl1, wg1, bg1,
              wo, bo, out_ref):
    x = jnp.concatenate([xw_ref[...], xn_ref[...]], axis=1)
    for wn, bn, wl, bl, wg, bg in ((wn0, bn0, wl0, bl0, wg0, bg0),
                                   (wn1, bn1, wl1, bl1, wg1, bg1)):
        gate = jax.nn.sigmoid(_mm(x, wg) + bg[...])
        nonlinear = jax.nn.relu(_mm(x, wn) + bn[...])
        linear = _mm(x, wl) + bl[...]
        x = gate * nonlinear + (1.0 - gate) * linear
    out_ref[...] = _mm(x, wo) + bo[...]


_mlp = pl.pallas_call(
    _mlp_body,
    out_shape=jax.ShapeDtypeStruct((B, CLASSES), jnp.float32),
)


def kernel(sequence, ngrams, word_table, ngram_table,
           Wn0, bn0, Wl0, bl0, Wg0, bg0,
           Wn1, bn1, Wl1, bl1, Wg1, bg1,
           Wo, bo):
    seq = sequence.astype(jnp.int32).reshape(B, NWIN, WIN)
    ngr = ngrams.astype(jnp.int32).reshape(B, NWIN, WIN)
    embw = _pool(seq, word_table)
    embn = _pool(ngr, ngram_table)
    return _mlp(embw, embn,
                Wn0, bn0.reshape(1, SIZE), Wl0, bl0.reshape(1, SIZE),
                Wg0, bg0.reshape(1, SIZE),
                Wn1, bn1.reshape(1, SIZE), Wl1, bl1.reshape(1, SIZE),
                Wg1, bg1.reshape(1, SIZE),
                Wo, bo.reshape(1, CLASSES))
